# Initial kernel scaffold; baseline (speedup 1.0000x reference)
import jax
import jax.numpy as jnp
from jax.experimental import pallas as pl


def kernel(x, edge_index, edge_attr, edge_weight, batch, W0, b0, W1, b1, W2, b2, Wm, bm):
    raise NotImplementedError("write your pallas kernel here")


# speedup vs baseline: 9.9860x; 9.9860x over previous
import functools

import jax
import jax.numpy as jnp
from jax import lax
from jax.experimental import pallas as pl
from jax.experimental.pallas import tpu as pltpu
from jax.experimental.pallas import tpu_sc as plsc

N_NODES = 10000
N_EDGES = 320000
D = 128
NUM_CLASSES = 10
NUM_GRAPHS = 128

NC = 2
NS = 16
NW = NC * NS

CHUNK = 128
EPT = N_EDGES // NW
NCH = -(-EPT // CHUNK)
EPT_PAD = NCH * CHUNK
E_PAD = EPT_PAD * NW
NPAD = 10240
SLAB = NPAD // NS

_f32 = jnp.float32
_i32 = jnp.int32

_mesh = plsc.VectorSubcoreMesh(
    core_axis_name="c", subcore_axis_name="s", num_cores=NC, num_subcores=NS)


def _deg_body(dst_hbm, ew_hbm, out_hbm, didx, ewv, zrow, degsh):
    c = lax.axis_index("c")
    s = lax.axis_index("s")
    wid = c * NS + s
    for f in range(SLAB // 16):
        zrow[pl.ds(f * 16, 16)] = jnp.zeros((16,), _f32)
    pltpu.sync_copy(zrow, degsh.at[pl.ds(s * SLAB, SLAB)])
    plsc.subcore_barrier()
    pltpu.sync_copy(dst_hbm.at[wid], didx)
    pltpu.sync_copy(ew_hbm.at[wid], ewv)

    def chunk(j, carry):
        pltpu.sync_copy(ewv.at[j], degsh.at[didx.at[j]], add=True)
        return carry

    lax.fori_loop(0, NCH, chunk, 0)
    plsc.subcore_barrier()
    pltpu.sync_copy(degsh.at[pl.ds(s * SLAB, SLAB)],
                    out_hbm.at[c, pl.ds(s * SLAB, SLAB)])


_deg_call = pl.kernel(
    _deg_body,
    out_type=jax.ShapeDtypeStruct((NC, NPAD), _f32),
    mesh=_mesh,
    scratch_types=[
        pltpu.VMEM((NCH, CHUNK), _i32),
        pltpu.VMEM((NCH, CHUNK), _f32),
        pltpu.VMEM((SLAB,), _f32),
        pltpu.VMEM_SHARED((NPAD,), _f32),
    ],
)


def _agg_body(src_hbm, dst_hbm, ew_hbm, hw2_hbm, out_hbm,
              sidx, didx, ewv, rows, accsh, sem):
    c = lax.axis_index("c")
    s = lax.axis_index("s")
    wid = c * NS + s

    def zrow_body(i, carry):
        for f in range(D // 16):
            rows[i, pl.ds(f * 16, 16)] = jnp.zeros((16,), _f32)
        return carry

    lax.fori_loop(0, CHUNK, zrow_body, 0)
    for r in range(SLAB // CHUNK):
        pltpu.sync_copy(rows, accsh.at[pl.ds(s * SLAB + r * CHUNK, CHUNK)])
    plsc.subcore_barrier()

    pltpu.sync_copy(src_hbm.at[wid], sidx)
    pltpu.sync_copy(dst_hbm.at[wid], didx)
    pltpu.sync_copy(ew_hbm.at[wid], ewv)

    def chunk(j, carry):
        pltpu.async_copy(hw2_hbm.at[sidx.at[j]], rows, sem).wait()

        def grp(t, c2):
            base = t * 16
            wv = ewv[j, pl.ds(base, 16)]
            for kk in range(16):
                w = wv[kk]
                for f in range(D // 16):
                    sl = pl.ds(f * 16, 16)
                    rows[base + kk, sl] = rows[base + kk, sl] * w
            return c2

        lax.fori_loop(0, CHUNK // 16, grp, 0)
        pltpu.sync_copy(rows, accsh.at[didx.at[j]], add=True)
        return carry

    lax.fori_loop(0, NCH, chunk, 0)
    plsc.subcore_barrier()
    pltpu.sync_copy(accsh.at[pl.ds(s * SLAB, SLAB)],
                    out_hbm.at[c, pl.ds(s * SLAB, SLAB)])


_agg_call = pl.kernel(
    _agg_body,
    out_type=jax.ShapeDtypeStruct((NC, NPAD, D), _f32),
    mesh=_mesh,
    scratch_types=[
        pltpu.VMEM((NCH, CHUNK), _i32),
        pltpu.VMEM((NCH, CHUNK), _i32),
        pltpu.VMEM((NCH, CHUNK), _f32),
        pltpu.VMEM((CHUNK, D), _f32),
        pltpu.VMEM_SHARED((NPAD, D), _f32),
        pltpu.SemaphoreType.DMA,
    ],
)


def _mmA_body(degT_ref, x_ref, w_ref, dinv_ref, hw2_ref):
    d = degT_ref[:, 0:1] + degT_ref[:, 1:2] + 1.0
    dinv = lax.rsqrt(d)
    dinv_ref[...] = dinv
    hw = jnp.dot(x_ref[...], w_ref[...], preferred_element_type=_f32)
    hw2_ref[...] = dinv[0:N_NODES, :] * hw


_mmA_call = pl.pallas_call(
    _mmA_body,
    out_shape=[
        jax.ShapeDtypeStruct((NPAD, 1), _f32),
        jax.ShapeDtypeStruct((N_NODES, D), _f32),
    ],
)


def _mmB_body(residual, acc_ref, hw2p_ref, hprev_ref, dinv_ref, b_ref, w_ref,
              h_ref, hw2_ref):
    agg = acc_ref[0, 0:N_NODES, :] + acc_ref[1, 0:N_NODES, :]
    dinv = dinv_ref[0:N_NODES, :]
    pre = dinv * (agg + hw2p_ref[...]) + b_ref[...]
    h = jnp.maximum(pre, 0.0)
    if residual:
        h = h + hprev_ref[...]
    h_ref[...] = h
    hw2_ref[...] = dinv * jnp.dot(h, w_ref[...], preferred_element_type=_f32)


def _make_mmB(residual):
    return pl.pallas_call(
        functools.partial(_mmB_body, residual),
        out_shape=[
            jax.ShapeDtypeStruct((N_NODES, D), _f32),
            jax.ShapeDtypeStruct((N_NODES, D), _f32),
        ],
    )


_mmB1 = _make_mmB(False)
_mmB2 = _make_mmB(True)


def _mmC_body(acc_ref, hw2p_ref, hprev_ref, dinv_ref, b_ref, batch_ref,
              wm_ref, bm_ref, out_ref):
    agg = acc_ref[0, 0:N_NODES, :] + acc_ref[1, 0:N_NODES, :]
    dinv = dinv_ref[0:N_NODES, :]
    h = jnp.maximum(dinv * (agg + hw2p_ref[...]) + b_ref[...], 0.0)
    h = h + hprev_ref[...]
    bb = jnp.broadcast_to(batch_ref[...], (NUM_GRAPHS, N_NODES))
    gids = lax.broadcasted_iota(_i32, (NUM_GRAPHS, N_NODES), 0)
    pt = (bb == gids).astype(_f32)
    sums = jnp.dot(pt, h, preferred_element_type=_f32)
    cnt = jnp.sum(pt, axis=1, keepdims=True)
    hg = sums / jnp.maximum(cnt, 1.0)
    out_ref[...] = jnp.dot(hg, wm_ref[...], preferred_element_type=_f32) \
        + bm_ref[...]


_mmC_call = pl.pallas_call(
    _mmC_body,
    out_shape=jax.ShapeDtypeStruct((NUM_GRAPHS, NUM_CLASSES), _f32),
)


def kernel(x, edge_index, edge_attr, edge_weight, batch,
           W0, b0, W1, b1, W2, b2, Wm, bm):
    src = edge_index[0].astype(_i32)
    dst = edge_index[1].astype(_i32)
    ew = edge_weight.astype(_f32)
    pad = E_PAD - N_EDGES
    srcp = jnp.concatenate([src, jnp.zeros((pad,), _i32)]).reshape(NW, NCH, CHUNK)
    dstp = jnp.concatenate([dst, jnp.zeros((pad,), _i32)]).reshape(NW, NCH, CHUNK)
    ewp = jnp.concatenate([ew, jnp.zeros((pad,), _f32)]).reshape(NW, NCH, CHUNK)

    degpair = _deg_call(dstp, ewp)
    dinv, hw2_0 = _mmA_call(degpair.T, x, W0)
    acc0 = _agg_call(srcp, dstp, ewp, hw2_0)
    h1, hw2_1 = _mmB1(acc0, hw2_0, hw2_0, dinv, b0.reshape(1, D), W1)
    acc1 = _agg_call(srcp, dstp, ewp, hw2_1)
    h2, hw2_2 = _mmB2(acc1, hw2_1, h1, dinv, b1.reshape(1, D), W2)
    acc2 = _agg_call(srcp, dstp, ewp, hw2_2)
    out = _mmC_call(acc2, hw2_2, h2, dinv, b2.reshape(1, D),
                    batch.astype(_i32).reshape(1, N_NODES),
                    Wm, bm.reshape(1, NUM_CLASSES))
    return out
